# baseline (speedup 1.0000x reference)
import jax
import jax.numpy as jnp
from jax.experimental import pallas as pl
from jax.experimental.pallas import tpu as pltpu

_CHUNK_ROWS = 5000
_DEPTH = 4
_SLOTS = 2 * _DEPTH


def _copy_body(src_ref, dst_ref, bufs, in_sems, out_sems):
    rows = src_ref.shape[0]
    nchunks = rows // _CHUNK_ROWS

    def in_copy(c, slot):
        return pltpu.make_async_copy(
            src_ref.at[pl.ds(c * _CHUNK_ROWS, _CHUNK_ROWS), :],
            bufs.at[slot],
            in_sems.at[slot],
        )

    def out_copy(c, slot):
        return pltpu.make_async_copy(
            bufs.at[slot],
            dst_ref.at[pl.ds(c * _CHUNK_ROWS, _CHUNK_ROWS), :],
            out_sems.at[slot],
        )

    for c in range(_DEPTH):
        in_copy(c, c % _SLOTS).start()

    for i in range(nchunks):
        slot = i % _SLOTS
        in_copy(i, slot).wait()
        out_copy(i, slot).start()
        nxt = i + _DEPTH
        if nxt < nchunks:
            nslot = nxt % _SLOTS
            if nxt >= _SLOTS:
                out_copy(nxt - _SLOTS, nslot).wait()
            in_copy(nxt, nslot).start()

    for k in range(min(_SLOTS, nchunks)):
        c = nchunks - min(_SLOTS, nchunks) + k
        out_copy(c, c % _SLOTS).wait()


def kernel(inputs, ent_embeds):
    del inputs
    rows, dim = ent_embeds.shape
    wide = ent_embeds.reshape(rows * dim // 128, 128)
    out = pl.pallas_call(
        _copy_body,
        out_shape=jax.ShapeDtypeStruct(wide.shape, wide.dtype),
        in_specs=[pl.BlockSpec(memory_space=pltpu.MemorySpace.HBM)],
        out_specs=pl.BlockSpec(memory_space=pltpu.MemorySpace.HBM),
        scratch_shapes=[
            pltpu.VMEM((_SLOTS, _CHUNK_ROWS, 128), jnp.float32),
            pltpu.SemaphoreType.DMA((_SLOTS,)),
            pltpu.SemaphoreType.DMA((_SLOTS,)),
        ],
    )(wide)
    return out.reshape(rows, dim)


# speedup vs baseline: 1.3016x; 1.3016x over previous
import jax
import jax.numpy as jnp
from jax import lax
from jax.experimental import pallas as pl
from jax.experimental.pallas import tpu as pltpu
from jax.experimental.pallas import tpu_sc as plsc

_NUM_CORES = 2
_NUM_SUBCORES = 16
_NUM_WORKERS = _NUM_CORES * _NUM_SUBCORES
_ROWS_PER_W = 31248
_CHUNK_ROWS = 248
_DEPTH = 2
_SLOTS = 2 * _DEPTH


def _sc_copy_body(src_hbm, dst_hbm, bufs, in_sems, out_sems):
    rows = src_hbm.shape[0]
    nchunks = _ROWS_PER_W // _CHUNK_ROWS
    wid = lax.axis_index("s") * _NUM_CORES + lax.axis_index("c")
    base = pl.multiple_of(wid * _ROWS_PER_W, 8)

    def in_copy(c, slot):
        return pltpu.make_async_copy(
            src_hbm.at[pl.ds(base + c * _CHUNK_ROWS, _CHUNK_ROWS), :],
            bufs.at[slot],
            in_sems.at[slot],
        )

    def out_copy(c, slot):
        return pltpu.make_async_copy(
            bufs.at[slot],
            dst_hbm.at[pl.ds(base + c * _CHUNK_ROWS, _CHUNK_ROWS), :],
            out_sems.at[slot],
        )

    for c in range(_DEPTH):
        in_copy(c, c % _SLOTS).start()

    for i in range(nchunks):
        slot = i % _SLOTS
        in_copy(i, slot).wait()
        out_copy(i, slot).start()
        nxt = i + _DEPTH
        if nxt < nchunks:
            nslot = nxt % _SLOTS
            if nxt >= _SLOTS:
                out_copy(nxt - _SLOTS, nslot).wait()
            in_copy(nxt, nslot).start()

    for k in range(min(_SLOTS, nchunks)):
        c = nchunks - min(_SLOTS, nchunks) + k
        out_copy(c, c % _SLOTS).wait()

    tail_rows = rows - _NUM_WORKERS * _ROWS_PER_W

    @pl.when(wid == _NUM_WORKERS - 1)
    def _copy_tail():
        tail_base = _NUM_WORKERS * _ROWS_PER_W
        stage = bufs.at[0, pl.ds(0, tail_rows), :]
        pltpu.sync_copy(src_hbm.at[pl.ds(tail_base, tail_rows), :], stage)
        pltpu.sync_copy(stage, dst_hbm.at[pl.ds(tail_base, tail_rows), :])


def kernel(inputs, ent_embeds):
    del inputs
    rows, dim = ent_embeds.shape
    mesh = plsc.VectorSubcoreMesh(
        core_axis_name="c",
        subcore_axis_name="s",
        num_cores=_NUM_CORES,
        num_subcores=_NUM_SUBCORES,
    )
    f = pl.kernel(
        _sc_copy_body,
        out_type=jax.ShapeDtypeStruct((rows, dim), ent_embeds.dtype),
        mesh=mesh,
        scratch_types=[
            pltpu.VMEM((_SLOTS, _CHUNK_ROWS, dim), jnp.float32),
            pltpu.SemaphoreType.DMA((_SLOTS,)),
            pltpu.SemaphoreType.DMA((_SLOTS,)),
        ],
    )
    return f(ent_embeds)


# speedup vs baseline: 7.7830x; 5.9797x over previous
import jax
import jax.numpy as jnp
from jax.experimental import pallas as pl
from jax.experimental.pallas import tpu as pltpu

_BLOCK_COLS = 8192


def _copy_body(src_ref, dst_ref):
    dst_ref[...] = src_ref[...]


def kernel(inputs, ent_embeds):
    del inputs
    rows, dim = ent_embeds.shape
    wide = ent_embeds.T
    grid = (rows + _BLOCK_COLS - 1) // _BLOCK_COLS
    out = pl.pallas_call(
        _copy_body,
        out_shape=jax.ShapeDtypeStruct(wide.shape, wide.dtype),
        grid=(grid,),
        in_specs=[pl.BlockSpec((dim, _BLOCK_COLS), lambda i: (0, i))],
        out_specs=pl.BlockSpec((dim, _BLOCK_COLS), lambda i: (0, i)),
    )(wide)
    return out.T


# speedup vs baseline: 8.5045x; 1.0927x over previous
import jax
import jax.numpy as jnp
from jax.experimental import pallas as pl
from jax.experimental.pallas import tpu as pltpu

_BLOCK_COLS = 16384


def _copy_body(src_ref, dst_ref):
    dst_ref[...] = src_ref[...]


def kernel(inputs, ent_embeds):
    del inputs
    rows, dim = ent_embeds.shape
    wide = ent_embeds.T
    grid = (rows + _BLOCK_COLS - 1) // _BLOCK_COLS
    out = pl.pallas_call(
        _copy_body,
        out_shape=jax.ShapeDtypeStruct(wide.shape, wide.dtype),
        grid=(grid,),
        in_specs=[pl.BlockSpec((dim, _BLOCK_COLS), lambda i: (0, i))],
        out_specs=pl.BlockSpec((dim, _BLOCK_COLS), lambda i: (0, i)),
    )(wide)
    return out.T


# speedup vs baseline: 8.6756x; 1.0201x over previous
import jax
import jax.numpy as jnp
from jax.experimental import pallas as pl
from jax.experimental.pallas import tpu as pltpu

_BLOCK_COLS = 32768


def _copy_body(src_ref, dst_ref):
    dst_ref[...] = src_ref[...]


def kernel(inputs, ent_embeds):
    del inputs
    rows, dim = ent_embeds.shape
    wide = ent_embeds.T
    grid = (rows + _BLOCK_COLS - 1) // _BLOCK_COLS
    out = pl.pallas_call(
        _copy_body,
        out_shape=jax.ShapeDtypeStruct(wide.shape, wide.dtype),
        grid=(grid,),
        in_specs=[pl.BlockSpec((dim, _BLOCK_COLS), lambda i: (0, i))],
        out_specs=pl.BlockSpec((dim, _BLOCK_COLS), lambda i: (0, i)),
    )(wide)
    return out.T
